# baseline (speedup 1.0000x reference)
import jax
import jax.numpy as jnp
from jax.experimental import pallas as pl


def _mm_kernel(x_ref, w_ref, o_ref):
    o_ref[...] = jnp.dot(x_ref[...], w_ref[...], preferred_element_type=jnp.float32)


def _matmul(x, w, bm=1000):
    m, k = x.shape
    _, n = w.shape
    grid = (m // bm,)
    return pl.pallas_call(
        _mm_kernel,
        grid=grid,
        in_specs=[
            pl.BlockSpec((bm, k), lambda i: (i, 0)),
            pl.BlockSpec((k, n), lambda i: (0, 0)),
        ],
        out_specs=pl.BlockSpec((bm, n), lambda i: (i, 0)),
        out_shape=jax.ShapeDtypeStruct((m, n), jnp.float32),
    )(x, w)


def _gatv2(x, src, dst, eattr, Wl, Wr, We, att, bias):
    n = x.shape[0]
    hid = Wl.shape[1]
    xlr = _matmul(x, jnp.concatenate([Wl, Wr], axis=1))
    xl = xlr[:, :hid]
    xr = xlr[:, hid:]
    ee = eattr @ We
    m = jax.nn.leaky_relu(xl[src] + xr[dst] + ee, negative_slope=0.2)
    logits = m @ att
    seg_max = jax.ops.segment_max(logits, dst, num_segments=n)
    seg_max = jnp.where(jnp.isfinite(seg_max), seg_max, 0.0)
    ex = jnp.exp(logits - seg_max[dst])
    denom = jax.ops.segment_sum(ex, dst, num_segments=n)
    alpha = ex / (denom[dst] + 1e-16)
    out = jax.ops.segment_sum(alpha[:, None] * xl[src], dst, num_segments=n)
    return out + bias


def kernel(x, edge_index, batch, edge_attr, Wl1, Wr1, We1, att1, b1, Wl2, Wr2, We2, att2, b2, fc1W, fc1b, gamma, beta, fc3W, fc3b):
    src = edge_index[0]
    dst = edge_index[1]
    h = jax.nn.elu(_gatv2(x, src, dst, edge_attr, Wl1, Wr1, We1, att1, b1))
    h = jax.nn.elu(_gatv2(h, src, dst, edge_attr, Wl2, Wr2, We2, att2, b2))
    G = 64
    sums = jax.ops.segment_sum(h, batch, num_segments=G)
    counts = jax.ops.segment_sum(jnp.ones((h.shape[0],), dtype=h.dtype), batch, num_segments=G)
    pooled = sums / jnp.maximum(counts, 1.0)[:, None]
    z = jax.nn.relu(pooled @ fc1W + fc1b)
    z = gamma * z / jnp.sqrt(1.0 + 1e-5) + beta
    out = z @ fc3W + fc3b
    return out


# speedup vs baseline: 3.6099x; 3.6099x over previous
import functools

import jax
import jax.numpy as jnp
from jax import lax
from jax.experimental import pallas as pl
from jax.experimental.pallas import tpu as pltpu
from jax.experimental.pallas import tpu_sc as plsc

N = 10000
E = 160000
G = 64
NC = 2
NS = 16
L = 16
BM = 1000
TW = 144

_MESH = plsc.VectorSubcoreMesh(
    core_axis_name="c", subcore_axis_name="s", num_cores=NC, num_subcores=NS)


def _mm1_body(x_ref, w_ref, oxl_ref, oxr_ref):
    res = jnp.dot(x_ref[...], w_ref[...], preferred_element_type=jnp.float32)
    oxl_ref[...] = res[:, :128]
    oxr_ref[...] = res[:, 128:]


def _matmul1(x, wcat):
    return pl.pallas_call(
        _mm1_body,
        grid=(N // BM, 2),
        in_specs=[
            pl.BlockSpec((BM, 256), lambda i, h: (i, 0)),
            pl.BlockSpec((256, 256), lambda i, h: (0, h)),
        ],
        out_specs=[
            pl.BlockSpec((BM, 128), lambda i, h: (i + h * (N // BM), 0)),
            pl.BlockSpec((BM, 128), lambda i, h: (i + h * (N // BM), 0)),
        ],
        out_shape=[
            jax.ShapeDtypeStruct((2 * N, 128), jnp.float32),
            jax.ShapeDtypeStruct((2 * N, 128), jnp.float32),
        ],
    )(x, wcat)


def _make_edge_ex(npairs):
    C = 128
    NCHUNK = E // C

    def body(atab, btab, src, dst, ea, attw, wew, ex_out,
             si, di, sihi, dihi, ev, lg, att_v, we_v,
             a0, b0, a1, b1, sems):
        wid = lax.axis_index("s") * NC + lax.axis_index("c")
        pltpu.sync_copy(attw, att_v)
        pltpu.sync_copy(wew, we_v)

        def chunk(j, _):
            k = j * (NC * NS) + wid
            pl.when(k < NCHUNK)(lambda: _chunk(k))
            return 0

        def _chunk(k):
            off = k * C
            pltpu.sync_copy(src.at[pl.ds(off, C)], si)
            pltpu.sync_copy(dst.at[pl.ds(off, C)], di)
            pltpu.sync_copy(ea.at[pl.ds(off, C)], ev)
            if npairs == 2:
                for g in range(C // L):
                    sl = pl.ds(g * L, L)
                    sihi[sl] = si[sl] + N
                    dihi[sl] = di[sl] + N
            cps = [pltpu.async_copy(atab.at[si], a0, sems.at[0]),
                   pltpu.async_copy(btab.at[di], b0, sems.at[1])]
            if npairs == 2:
                cps.append(pltpu.async_copy(atab.at[sihi], a1, sems.at[2]))
                cps.append(pltpu.async_copy(btab.at[dihi], b1, sems.at[3]))
            for cp in cps:
                cp.wait()

            abufs = [a0, a1][:npairs]
            bbufs = [b0, b1][:npairs]
            lane = lax.iota(jnp.int32, L)

            def grp(g, _):
                evv = ev[pl.ds(g * L, L)]
                lgv = jnp.zeros((L,), jnp.float32)
                for t in range(L):
                    e = g * L + t
                    ea_e = evv[t]
                    acc = jnp.zeros((L,), jnp.float32)
                    for p in range(npairs):
                        for k in range(128 // L):
                            sl = pl.ds(k * L, L)
                            slw = pl.ds(p * 128 + k * L, L)
                            v = (abufs[p][e, sl] + bbufs[p][e, sl]
                                 + ea_e * we_v[slw])
                            m = jnp.where(v >= 0.0, v, 0.2 * v)
                            acc = acc + att_v[slw] * m
                    for sh in (8, 4, 2, 1):
                        acc = acc + acc.at[lane ^ sh].get(
                            mode="promise_in_bounds")
                    onehot = jnp.where(lane == t, 1.0, 0.0)
                    lgv = lgv + acc * onehot
                lg[pl.ds(g * L, L)] = lgv
                return 0

            lax.fori_loop(0, C // L, grp, 0)

            def expg(g, _):
                sl = pl.ds(g * L, L)
                lg[sl] = jnp.exp(lg[sl])
                return 0

            lax.fori_loop(0, C // L, expg, 0)
            pltpu.sync_copy(lg, ex_out.at[pl.ds(off, C)])

        lax.fori_loop(0, (NCHUNK + NC * NS - 1) // (NC * NS), chunk, 0)

    fwidth = npairs * 128
    scratch = [
        pltpu.VMEM((C,), jnp.int32),
        pltpu.VMEM((C,), jnp.int32),
        pltpu.VMEM((C,), jnp.int32),
        pltpu.VMEM((C,), jnp.int32),
        pltpu.VMEM((C,), jnp.float32),
        pltpu.VMEM((C,), jnp.float32),
        pltpu.VMEM((fwidth,), jnp.float32),
        pltpu.VMEM((fwidth,), jnp.float32),
        pltpu.VMEM((C, 128), jnp.float32),
        pltpu.VMEM((C, 128), jnp.float32),
        pltpu.VMEM((C, 128), jnp.float32),
        pltpu.VMEM((C, 128), jnp.float32),
        pltpu.SemaphoreType.DMA((4,)),
    ]
    return pl.kernel(
        body,
        out_type=jax.ShapeDtypeStruct((E,), jnp.float32),
        mesh=_MESH,
        scratch_types=scratch,
    )


def _make_accum(split_edges, idx_core_off, C):
    NCHUNK = E // C
    RPT = N // NS
    ZR = 25

    def body(tab, src, dst, ex, t_out, si, siadj, di, exv, a0, rows, zbuf,
             t_sp, sems):
        c = lax.axis_index("c")
        s = lax.axis_index("s")

        def zrow(r, _):
            for k in range(TW // L):
                zbuf[r, pl.ds(k * L, L)] = jnp.zeros((L,), jnp.float32)
            return 0

        lax.fori_loop(0, ZR, zrow, 0)
        for z in range(RPT // ZR):
            pltpu.sync_copy(zbuf, t_sp.at[pl.ds(s * RPT + z * ZR, ZR)])
        plsc.subcore_barrier()

        lane = lax.iota(jnp.int32, L)
        onehot0 = jnp.where(lane == 0, 1.0, 0.0)

        owner = s if split_edges is False else s * NC + c
        nown = NS if split_edges is False else NC * NS

        def chunk(j, _):
            k = j * nown + owner
            pl.when(k < NCHUNK)(lambda: _chunk(k))
            return 0

        def _chunk(k):
            off = k * C
            pltpu.sync_copy(src.at[pl.ds(off, C)], si)
            pltpu.sync_copy(dst.at[pl.ds(off, C)], di)
            pltpu.sync_copy(ex.at[pl.ds(off, C)], exv)
            for g in range(C // L):
                sl = pl.ds(g * L, L)
                siadj[sl] = si[sl] + c * idx_core_off
            pltpu.async_copy(tab.at[siadj], a0, sems.at[0]).wait()

            def grp(g, _):
                exvv = exv[pl.ds(g * L, L)]
                for t in range(L):
                    e = g * L + t
                    x = exvv[t]
                    rows[e, pl.ds(0, L)] = x * onehot0
                    for k2 in range(128 // L):
                        rows[e, pl.ds(L + k2 * L, L)] = (
                            x * a0[e, pl.ds(k2 * L, L)])
                return 0

            lax.fori_loop(0, C // L, grp, 0)
            pltpu.sync_copy(rows, t_sp.at[di], add=True)

        lax.fori_loop(0, (NCHUNK + nown - 1) // nown, chunk, 0)
        plsc.subcore_barrier()
        pltpu.sync_copy(t_sp.at[pl.ds(s * RPT, RPT)],
                        t_out.at[pl.ds(c * N + s * RPT, RPT)])

    scratch = [
        pltpu.VMEM((C,), jnp.int32),
        pltpu.VMEM((C,), jnp.int32),
        pltpu.VMEM((C,), jnp.int32),
        pltpu.VMEM((C,), jnp.float32),
        pltpu.VMEM((C, 128), jnp.float32),
        pltpu.VMEM((C, TW), jnp.float32),
        pltpu.VMEM((ZR, TW), jnp.float32),
        pltpu.VMEM_SHARED((N, TW), jnp.float32),
        pltpu.SemaphoreType.DMA((1,)),
    ]
    return pl.kernel(
        body,
        out_type=jax.ShapeDtypeStruct((2 * N, TW), jnp.float32),
        mesh=_MESH,
        scratch_types=scratch,
        compiler_params=pltpu.CompilerParams(use_tc_tiling_on_sc=False),
    )


def _mid_body(ta_ref, tb_ref, w_ref, b_ref, oxl_ref, oxr_ref):
    den = ta_ref[:, 0:1] + 1e-16
    hlo = ta_ref[:, 16:] / den
    hhi = tb_ref[:, 16:] / den
    h = jnp.concatenate([hlo, hhi], axis=1) + b_ref[...]
    h = jnp.where(h > 0.0, h, jnp.exp(jnp.minimum(h, 0.0)) - 1.0)
    res = jnp.dot(h, w_ref[...], preferred_element_type=jnp.float32)
    oxl_ref[...] = res[:, :128]
    oxr_ref[...] = res[:, 128:]


def _mid(t1, w2cat, b1):
    return pl.pallas_call(
        _mid_body,
        grid=(N // BM,),
        in_specs=[
            pl.BlockSpec((BM, TW), lambda i: (i, 0)),
            pl.BlockSpec((BM, TW), lambda i: (i + N // BM, 0)),
            pl.BlockSpec((256, 256), lambda i: (0, 0)),
            pl.BlockSpec((1, 256), lambda i: (0, 0)),
        ],
        out_specs=[
            pl.BlockSpec((BM, 128), lambda i: (i, 0)),
            pl.BlockSpec((BM, 128), lambda i: (i, 0)),
        ],
        out_shape=[
            jax.ShapeDtypeStruct((N, 128), jnp.float32),
            jax.ShapeDtypeStruct((N, 128), jnp.float32),
        ],
    )(t1, t1, w2cat, b1.reshape(1, 256))


_BN_INV = 0.9999950000374997


def _final_body(ta_ref, tb_ref, batch_ref, b2_ref, fc1w_ref, fc1b_ref,
                gamma_ref, beta_ref, fc3w_ref, fc3b_ref, out_ref,
                sums_ref, cnt_ref):
    i = pl.program_id(0)

    @pl.when(i == 0)
    def _():
        sums_ref[...] = jnp.zeros_like(sums_ref)
        cnt_ref[...] = jnp.zeros_like(cnt_ref)

    t = ta_ref[...] + tb_ref[...]
    den = t[:, 0:1] + 1e-16
    h = t[:, 16:] / den + b2_ref[...]
    h = jnp.where(h > 0.0, h, jnp.exp(jnp.minimum(h, 0.0)) - 1.0)
    ids = lax.broadcasted_iota(jnp.int32, (G, BM), 0)
    onehot = (ids == batch_ref[0]).astype(jnp.float32)
    sums_ref[...] += jnp.dot(onehot, h, preferred_element_type=jnp.float32)
    cnt_ref[...] += jnp.sum(onehot, axis=1, keepdims=True)

    @pl.when(i == N // BM - 1)
    def _():
        pooled = sums_ref[...] / jnp.maximum(cnt_ref[:, 0:1], 1.0)
        z = jnp.dot(pooled, fc1w_ref[...], preferred_element_type=jnp.float32)
        z = jnp.maximum(z + fc1b_ref[...], 0.0)
        z = gamma_ref[...] * z * _BN_INV + beta_ref[...]
        out_ref[...] = (
            jnp.dot(z, fc3w_ref[...], preferred_element_type=jnp.float32)
            + fc3b_ref[...])


def _final(t2, batch2d, b2, fc1W, fc1b, gamma, beta, fc3W, fc3b):
    return pl.pallas_call(
        _final_body,
        grid=(N // BM,),
        in_specs=[
            pl.BlockSpec((BM, TW), lambda i: (i, 0)),
            pl.BlockSpec((BM, TW), lambda i: (i + N // BM, 0)),
            pl.BlockSpec((1, 1, BM), lambda i: (i, 0, 0)),
            pl.BlockSpec((1, 128), lambda i: (0, 0)),
            pl.BlockSpec((128, 128), lambda i: (0, 0)),
            pl.BlockSpec((1, 128), lambda i: (0, 0)),
            pl.BlockSpec((1, 128), lambda i: (0, 0)),
            pl.BlockSpec((1, 128), lambda i: (0, 0)),
            pl.BlockSpec((128, 1), lambda i: (0, 0)),
            pl.BlockSpec((1, 1), lambda i: (0, 0)),
        ],
        out_specs=pl.BlockSpec((G, 1), lambda i: (0, 0)),
        out_shape=jax.ShapeDtypeStruct((G, 1), jnp.float32),
        scratch_shapes=[
            pltpu.VMEM((G, 128), jnp.float32),
            pltpu.VMEM((G, 1), jnp.float32),
        ],
    )(t2, t2, batch2d, b2.reshape(1, 128), fc1W, fc1b.reshape(1, 128),
      gamma.reshape(1, 128), beta.reshape(1, 128), fc3W, fc3b.reshape(1, 1))


_edge_ex2 = _make_edge_ex(2)
_edge_ex1 = _make_edge_ex(1)
_accum_l1 = _make_accum(split_edges=False, idx_core_off=N, C=64)
_accum_l2 = _make_accum(split_edges=True, idx_core_off=0, C=64)


def kernel(x, edge_index, batch, edge_attr, Wl1, Wr1, We1, att1, b1,
           Wl2, Wr2, We2, att2, b2, fc1W, fc1b, gamma, beta, fc3W, fc3b):
    src = edge_index[0]
    dst = edge_index[1]
    ea = edge_attr[:, 0]

    wcat1 = jnp.concatenate(
        [Wl1[:, :128], Wr1[:, :128], Wl1[:, 128:], Wr1[:, 128:]], axis=1)
    xl1, xr1 = _matmul1(x, wcat1)

    ex1 = _edge_ex2(xl1, xr1, src, dst, ea, att1, We1[0])
    t1 = _accum_l1(xl1, src, dst, ex1)

    w2cat = jnp.concatenate([Wl2, Wr2], axis=1)
    xl2, xr2 = _mid(t1, w2cat, b1)

    ex2 = _edge_ex1(xl2, xr2, src, dst, ea, att2, We2[0])
    t2 = _accum_l2(xl2, src, dst, ex2)

    return _final(t2, batch.reshape(N // BM, 1, BM), b2, fc1W, fc1b,
                  gamma, beta, fc3W, fc3b)


# speedup vs baseline: 6.1430x; 1.7017x over previous
import jax
import jax.numpy as jnp
from jax import lax
from jax.experimental import pallas as pl
from jax.experimental.pallas import tpu as pltpu
from jax.experimental.pallas import tpu_sc as plsc

N = 10000
E = 160000
G = 64
NC = 2
NS = 16
NW = NC * NS
L = 16
BM = 1000

_MESH = plsc.VectorSubcoreMesh(
    core_axis_name="c", subcore_axis_name="s", num_cores=NC, num_subcores=NS)


def _mm1_body(x_ref, w_ref, oxl_ref, oxr_ref):
    res = jnp.dot(x_ref[...], w_ref[...], preferred_element_type=jnp.float32)
    oxl_ref[...] = res[:, :128]
    oxr_ref[...] = res[:, 128:]


def _matmul1(x, wcat):
    return pl.pallas_call(
        _mm1_body,
        grid=(N // BM, 2),
        in_specs=[
            pl.BlockSpec((BM, 256), lambda i, h: (i, 0)),
            pl.BlockSpec((256, 256), lambda i, h: (0, h)),
        ],
        out_specs=[
            pl.BlockSpec((BM, 128), lambda i, h: (i + h * (N // BM), 0)),
            pl.BlockSpec((BM, 128), lambda i, h: (i + h * (N // BM), 0)),
        ],
        out_shape=[
            jax.ShapeDtypeStruct((2 * N, 128), jnp.float32),
            jax.ShapeDtypeStruct((2 * N, 128), jnp.float32),
        ],
    )(x, wcat)


def _make_edge_ex(npairs):
    C = 80
    NCHUNK = E // C
    NJ2 = (NCHUNK // NW + 2) // 2

    def body(atab, btab, src, dst, ea, attw, wew, ex_out, *refs):
        nper = 10
        sets = [refs[i * nper:(i + 1) * nper] for i in range(2)]
        att_v, we_v, sems = refs[2 * nper:]
        wid = lax.axis_index("s") * NC + lax.axis_index("c")
        pltpu.sync_copy(attw, att_v)
        pltpu.sync_copy(wew, we_v)
        lane = lax.iota(jnp.int32, L)

        def gathers(s):
            si, di, sihi, dihi = sets[s][:4]
            a0, b0, a1, b1 = sets[s][6:]
            gl = [(atab.at[si], a0), (btab.at[di], b0)]
            if npairs == 2:
                gl += [(atab.at[sihi], a1), (btab.at[dihi], b1)]
            return gl

        def issue(k, s):
            si, di, sihi, dihi, ev = sets[s][:5]
            off = k * C
            pltpu.sync_copy(src.at[pl.ds(off, C)], si)
            pltpu.sync_copy(dst.at[pl.ds(off, C)], di)
            pltpu.sync_copy(ea.at[pl.ds(off, C)], ev)
            if npairs == 2:
                for g in range(C // L):
                    sl = pl.ds(g * L, L)
                    sihi[sl] = si[sl] + N
                    dihi[sl] = di[sl] + N
            for i, (sr, db) in enumerate(gathers(s)):
                pltpu.async_copy(sr, db, sems.at[4 * s + i])

        def compute(k, s):
            ev, lg = sets[s][4:6]
            a0, b0, a1, b1 = sets[s][6:]
            for i, (sr, db) in enumerate(gathers(s)):
                pltpu.make_async_copy(sr, db, sems.at[4 * s + i]).wait()
            abufs = [a0, a1][:npairs]
            bbufs = [b0, b1][:npairs]

            def grp(g, _):
                evv = ev[pl.ds(g * L, L)]
                lgv = jnp.zeros((L,), jnp.float32)
                for t in range(L):
                    e = g * L + t
                    ea_e = evv[t]
                    acc = jnp.zeros((L,), jnp.float32)
                    for p in range(npairs):
                        for kk in range(128 // L):
                            sl = pl.ds(kk * L, L)
                            slw = pl.ds(p * 128 + kk * L, L)
                            v = (abufs[p][e, sl] + bbufs[p][e, sl]
                                 + ea_e * we_v[slw])
                            m = jnp.where(v >= 0.0, v, 0.2 * v)
                            acc = acc + att_v[slw] * m
                    for sh in (8, 4, 2, 1):
                        acc = acc + acc.at[lane ^ sh].get(
                            mode="promise_in_bounds")
                    onehot = jnp.where(lane == t, 1.0, 0.0)
                    lgv = lgv + acc * onehot
                lg[pl.ds(g * L, L)] = lgv
                return 0

            lax.fori_loop(0, C // L, grp, 0)

            def expg(g, _):
                sl = pl.ds(g * L, L)
                lg[sl] = jnp.exp(lg[sl])
                return 0

            lax.fori_loop(0, C // L, expg, 0)
            pltpu.sync_copy(lg, ex_out.at[pl.ds(k * C, C)])

        pl.when(wid < NCHUNK)(lambda: issue(wid, 0))

        def outer(jj, _):
            for b in (0, 1):
                k = (2 * jj + b) * NW + wid
                kn = k + NW
                pl.when(kn < NCHUNK)(lambda kn=kn, b=b: issue(kn, 1 - b))
                pl.when(k < NCHUNK)(lambda k=k, b=b: compute(k, b))
            return 0

        lax.fori_loop(0, NJ2, outer, 0)

    def one_set():
        return [
            pltpu.VMEM((C,), jnp.int32),
            pltpu.VMEM((C,), jnp.int32),
            pltpu.VMEM((C,), jnp.int32),
            pltpu.VMEM((C,), jnp.int32),
            pltpu.VMEM((C,), jnp.float32),
            pltpu.VMEM((C,), jnp.float32),
            pltpu.VMEM((C, 128), jnp.float32),
            pltpu.VMEM((C, 128), jnp.float32),
            pltpu.VMEM((C, 128), jnp.float32),
            pltpu.VMEM((C, 128), jnp.float32),
        ]

    fwidth = npairs * 128
    scratch = one_set() + one_set() + [
        pltpu.VMEM((fwidth,), jnp.float32),
        pltpu.VMEM((fwidth,), jnp.float32),
        pltpu.SemaphoreType.DMA((8,)),
    ]
    return pl.kernel(
        body,
        out_type=jax.ShapeDtypeStruct((E,), jnp.float32),
        mesh=_MESH,
        scratch_types=scratch,
    )


def _make_accum(split_edges, idx_core_off, C):
    NCHUNK = E // C
    RPT = N // NS
    ZR = 25

    def body(tab, src, dst, ex, tf_out, td_out, *refs):
        nper = 6
        sets = [refs[i * nper:(i + 1) * nper] for i in range(2)]
        zbuf, zbufd, t_spf, t_spd, sems = refs[2 * nper:]
        c = lax.axis_index("c")
        s = lax.axis_index("s")
        owner = s * NC + c if split_edges else s
        nown = NW if split_edges else NS
        nj2 = (NCHUNK // nown + 2) // 2

        def zrow(r, _):
            for kk in range(128 // L):
                zbuf[r, pl.ds(kk * L, L)] = jnp.zeros((L,), jnp.float32)
            zbufd[r, pl.ds(0, L)] = jnp.zeros((L,), jnp.float32)
            return 0

        lax.fori_loop(0, ZR, zrow, 0)
        for z in range(RPT // ZR):
            pltpu.sync_copy(zbuf, t_spf.at[pl.ds(s * RPT + z * ZR, ZR)])
            pltpu.sync_copy(zbufd, t_spd.at[pl.ds(s * RPT + z * ZR, ZR)])
        plsc.subcore_barrier()

        lane = lax.iota(jnp.int32, L)
        onehot0 = jnp.where(lane == 0, 1.0, 0.0)

        def drain(s_):
            si, siadj, di, exv, a0, exb = sets[s_]
            pltpu.make_async_copy(a0, t_spf.at[di], sems.at[2 + 2 * s_]).wait()
            pltpu.make_async_copy(exb, t_spd.at[di], sems.at[3 + 2 * s_]).wait()

        def issue(k, s_):
            si, siadj, di, exv, a0, exb = sets[s_]
            pl.when(k >= 2 * nown)(lambda: drain(s_))
            off = k * C
            pltpu.sync_copy(src.at[pl.ds(off, C)], si)
            pltpu.sync_copy(dst.at[pl.ds(off, C)], di)
            pltpu.sync_copy(ex.at[pl.ds(off, C)], exv)
            for g in range(C // L):
                sl = pl.ds(g * L, L)
                siadj[sl] = si[sl] + c * idx_core_off
            pltpu.async_copy(tab.at[siadj], a0, sems.at[s_])

        def compute(k, s_):
            si, siadj, di, exv, a0, exb = sets[s_]
            pltpu.make_async_copy(tab.at[siadj], a0, sems.at[s_]).wait()

            def grp(g, _):
                exvv = exv[pl.ds(g * L, L)]
                for t in range(L):
                    e = g * L + t
                    x = exvv[t]
                    exb[e, pl.ds(0, L)] = x * onehot0
                    for kk in range(128 // L):
                        sl = pl.ds(kk * L, L)
                        a0[e, sl] = x * a0[e, sl]
                return 0

            lax.fori_loop(0, C // L, grp, 0)
            pltpu.async_copy(a0, t_spf.at[di], sems.at[2 + 2 * s_], add=True)
            pltpu.async_copy(exb, t_spd.at[di], sems.at[3 + 2 * s_], add=True)

        pl.when(owner < NCHUNK)(lambda: issue(owner, 0))

        def outer(jj, _):
            for b in (0, 1):
                k = (2 * jj + b) * nown + owner
                kn = k + nown
                pl.when(kn < NCHUNK)(lambda kn=kn, b=b: issue(kn, 1 - b))
                pl.when(k < NCHUNK)(lambda k=k, b=b: compute(k, b))
            return 0

        lax.fori_loop(0, nj2, outer, 0)
        drain(0)
        drain(1)
        plsc.subcore_barrier()
        pltpu.sync_copy(t_spf.at[pl.ds(s * RPT, RPT)],
                        tf_out.at[pl.ds(c * N + s * RPT, RPT)])
        pltpu.sync_copy(t_spd.at[pl.ds(s * RPT, RPT)],
                        td_out.at[pl.ds(c * N + s * RPT, RPT)])

    def one_set():
        return [
            pltpu.VMEM((C,), jnp.int32),
            pltpu.VMEM((C,), jnp.int32),
            pltpu.VMEM((C,), jnp.int32),
            pltpu.VMEM((C,), jnp.float32),
            pltpu.VMEM((C, 128), jnp.float32),
            pltpu.VMEM((C, L), jnp.float32),
        ]

    scratch = one_set() + one_set() + [
        pltpu.VMEM((ZR, 128), jnp.float32),
        pltpu.VMEM((ZR, L), jnp.float32),
        pltpu.VMEM_SHARED((N, 128), jnp.float32),
        pltpu.VMEM_SHARED((N, L), jnp.float32),
        pltpu.SemaphoreType.DMA((6,)),
    ]
    return pl.kernel(
        body,
        out_type=[jax.ShapeDtypeStruct((2 * N, 128), jnp.float32),
                  jax.ShapeDtypeStruct((2 * N, L), jnp.float32)],
        mesh=_MESH,
        scratch_types=scratch,
        compiler_params=pltpu.CompilerParams(use_tc_tiling_on_sc=False),
    )


def _mid_body(tfa_ref, tfb_ref, td_ref, w_ref, b_ref, oxl_ref, oxr_ref):
    den = td_ref[:, 0:1] + 1e-16
    hlo = tfa_ref[...] / den
    hhi = tfb_ref[...] / den
    h = jnp.concatenate([hlo, hhi], axis=1) + b_ref[...]
    h = jnp.where(h > 0.0, h, jnp.exp(jnp.minimum(h, 0.0)) - 1.0)
    res = jnp.dot(h, w_ref[...], preferred_element_type=jnp.float32)
    oxl_ref[...] = res[:, :128]
    oxr_ref[...] = res[:, 128:]


def _mid(tf, td, w2cat, b1):
    return pl.pallas_call(
        _mid_body,
        grid=(N // BM,),
        in_specs=[
            pl.BlockSpec((BM, 128), lambda i: (i, 0)),
            pl.BlockSpec((BM, 128), lambda i: (i + N // BM, 0)),
            pl.BlockSpec((BM, L), lambda i: (i, 0)),
            pl.BlockSpec((256, 256), lambda i: (0, 0)),
            pl.BlockSpec((1, 256), lambda i: (0, 0)),
        ],
        out_specs=[
            pl.BlockSpec((BM, 128), lambda i: (i, 0)),
            pl.BlockSpec((BM, 128), lambda i: (i, 0)),
        ],
        out_shape=[
            jax.ShapeDtypeStruct((N, 128), jnp.float32),
            jax.ShapeDtypeStruct((N, 128), jnp.float32),
        ],
    )(tf, tf, td, w2cat, b1.reshape(1, 256))


_BN_INV = 0.9999950000374997


def _final_body(tfa_ref, tfb_ref, tda_ref, tdb_ref, batch_ref, b2_ref,
                fc1w_ref, fc1b_ref, gamma_ref, beta_ref, fc3w_ref, fc3b_ref,
                out_ref, sums_ref, cnt_ref):
    i = pl.program_id(0)

    @pl.when(i == 0)
    def _():
        sums_ref[...] = jnp.zeros_like(sums_ref)
        cnt_ref[...] = jnp.zeros_like(cnt_ref)

    den = tda_ref[:, 0:1] + tdb_ref[:, 0:1] + 1e-16
    h = (tfa_ref[...] + tfb_ref[...]) / den + b2_ref[...]
    h = jnp.where(h > 0.0, h, jnp.exp(jnp.minimum(h, 0.0)) - 1.0)
    ids = lax.broadcasted_iota(jnp.int32, (G, BM), 0)
    onehot = (ids == batch_ref[0]).astype(jnp.float32)
    sums_ref[...] += jnp.dot(onehot, h, preferred_element_type=jnp.float32)
    cnt_ref[...] += jnp.sum(onehot, axis=1, keepdims=True)

    @pl.when(i == N // BM - 1)
    def _():
        pooled = sums_ref[...] / jnp.maximum(cnt_ref[:, 0:1], 1.0)
        z = jnp.dot(pooled, fc1w_ref[...], preferred_element_type=jnp.float32)
        z = jnp.maximum(z + fc1b_ref[...], 0.0)
        z = gamma_ref[...] * z * _BN_INV + beta_ref[...]
        out_ref[...] = (
            jnp.dot(z, fc3w_ref[...], preferred_element_type=jnp.float32)
            + fc3b_ref[...])


def _final(tf2, td2, batch3d, b2, fc1W, fc1b, gamma, beta, fc3W, fc3b):
    return pl.pallas_call(
        _final_body,
        grid=(N // BM,),
        in_specs=[
            pl.BlockSpec((BM, 128), lambda i: (i, 0)),
            pl.BlockSpec((BM, 128), lambda i: (i + N // BM, 0)),
            pl.BlockSpec((BM, L), lambda i: (i, 0)),
            pl.BlockSpec((BM, L), lambda i: (i + N // BM, 0)),
            pl.BlockSpec((1, 1, BM), lambda i: (i, 0, 0)),
            pl.BlockSpec((1, 128), lambda i: (0, 0)),
            pl.BlockSpec((128, 128), lambda i: (0, 0)),
            pl.BlockSpec((1, 128), lambda i: (0, 0)),
            pl.BlockSpec((1, 128), lambda i: (0, 0)),
            pl.BlockSpec((1, 128), lambda i: (0, 0)),
            pl.BlockSpec((128, 1), lambda i: (0, 0)),
            pl.BlockSpec((1, 1), lambda i: (0, 0)),
        ],
        out_specs=pl.BlockSpec((G, 1), lambda i: (0, 0)),
        out_shape=jax.ShapeDtypeStruct((G, 1), jnp.float32),
        scratch_shapes=[
            pltpu.VMEM((G, 128), jnp.float32),
            pltpu.VMEM((G, 1), jnp.float32),
        ],
    )(tf2, tf2, td2, td2, batch3d, b2.reshape(1, 128), fc1W,
      fc1b.reshape(1, 128), gamma.reshape(1, 128), beta.reshape(1, 128),
      fc3W, fc3b.reshape(1, 1))


_edge_ex2 = _make_edge_ex(2)
_edge_ex1 = _make_edge_ex(1)
_accum_l1 = _make_accum(split_edges=False, idx_core_off=N, C=64)
_accum_l2 = _make_accum(split_edges=True, idx_core_off=0, C=64)


def kernel(x, edge_index, batch, edge_attr, Wl1, Wr1, We1, att1, b1,
           Wl2, Wr2, We2, att2, b2, fc1W, fc1b, gamma, beta, fc3W, fc3b):
    src = edge_index[0]
    dst = edge_index[1]
    ea = edge_attr[:, 0]

    wcat1 = jnp.concatenate(
        [Wl1[:, :128], Wr1[:, :128], Wl1[:, 128:], Wr1[:, 128:]], axis=1)
    xl1, xr1 = _matmul1(x, wcat1)

    ex1 = _edge_ex2(xl1, xr1, src, dst, ea, att1, We1[0])
    tf1, td1 = _accum_l1(xl1, src, dst, ex1)

    w2cat = jnp.concatenate([Wl2, Wr2], axis=1)
    xl2, xr2 = _mid(tf1, td1, w2cat, b1)

    ex2 = _edge_ex1(xl2, xr2, src, dst, ea, att2, We2[0])
    tf2, td2 = _accum_l2(xl2, src, dst, ex2)

    return _final(tf2, td2, batch.reshape(N // BM, 1, BM), b2, fc1W, fc1b,
                  gamma, beta, fc3W, fc3b)


# speedup vs baseline: 7.3790x; 1.2012x over previous
import jax
import jax.numpy as jnp
from jax import lax
from jax.experimental import pallas as pl
from jax.experimental.pallas import tpu as pltpu
from jax.experimental.pallas import tpu_sc as plsc

N = 10000
E = 160000
G = 64
NC = 2
NS = 16
NW = NC * NS
L = 16
BM = 1000

_MESH = plsc.VectorSubcoreMesh(
    core_axis_name="c", subcore_axis_name="s", num_cores=NC, num_subcores=NS)


def _mm1_body(x_ref, w_ref, oxl_ref, oxr_ref):
    res = jnp.dot(x_ref[...], w_ref[...], preferred_element_type=jnp.float32)
    oxl_ref[...] = res[:, :128]
    oxr_ref[...] = res[:, 128:]


def _matmul1(x, wcat):
    return pl.pallas_call(
        _mm1_body,
        grid=(N // BM, 2),
        in_specs=[
            pl.BlockSpec((BM, 256), lambda i, h: (i, 0)),
            pl.BlockSpec((256, 256), lambda i, h: (0, h)),
        ],
        out_specs=[
            pl.BlockSpec((BM, 128), lambda i, h: (i + h * (N // BM), 0)),
            pl.BlockSpec((BM, 128), lambda i, h: (i + h * (N // BM), 0)),
        ],
        out_shape=[
            jax.ShapeDtypeStruct((2 * N, 128), jnp.float32),
            jax.ShapeDtypeStruct((2 * N, 128), jnp.float32),
        ],
    )(x, wcat)


def _make_edge_ex(npairs):
    C = 80
    NCHUNK = E // C
    NJ2 = (NCHUNK // NW + 2) // 2

    def body(atab, btab, edata, ea, attw, wew, ex_out, *refs):
        nper = 9
        sets = [refs[i * nper:(i + 1) * nper] for i in range(2)]
        att_v, we_v, sems = refs[2 * nper:]
        wid = lax.axis_index("s") * NC + lax.axis_index("c")
        pltpu.sync_copy(attw, att_v)
        pltpu.sync_copy(wew, we_v)
        lane = lax.iota(jnp.int32, L)

        def gathers(s):
            eb, sihi, dihi = sets[s][:3]
            a0, b0, a1, b1 = sets[s][5:]
            gl = [(atab.at[eb.at[0]], a0), (btab.at[eb.at[1]], b0)]
            if npairs == 2:
                gl += [(atab.at[sihi], a1), (btab.at[dihi], b1)]
            return gl

        def issue(k, s):
            eb, sihi, dihi, ev = sets[s][:4]
            off = k * C
            pltpu.sync_copy(edata.at[:, pl.ds(off, C)], eb)
            pltpu.sync_copy(ea.at[pl.ds(off, C)], ev)
            if npairs == 2:
                for g in range(C // L):
                    sl = pl.ds(g * L, L)
                    sihi[sl] = eb[0, sl] + N
                    dihi[sl] = eb[1, sl] + N
            for i, (sr, db) in enumerate(gathers(s)):
                pltpu.async_copy(sr, db, sems.at[4 * s + i])

        def compute(k, s):
            ev, lg = sets[s][3:5]
            a0, b0, a1, b1 = sets[s][5:]
            for i, (sr, db) in enumerate(gathers(s)):
                pltpu.make_async_copy(sr, db, sems.at[4 * s + i]).wait()
            abufs = [a0, a1][:npairs]
            bbufs = [b0, b1][:npairs]

            def grp(g, _):
                evv = ev[pl.ds(g * L, L)]
                lgv = jnp.zeros((L,), jnp.float32)
                for t in range(L):
                    e = g * L + t
                    ea_e = evv[t]
                    acc = jnp.zeros((L,), jnp.float32)
                    for p in range(npairs):
                        for kk in range(128 // L):
                            sl = pl.ds(kk * L, L)
                            slw = pl.ds(p * 128 + kk * L, L)
                            v = (abufs[p][e, sl] + bbufs[p][e, sl]
                                 + ea_e * we_v[slw])
                            m = jnp.where(v >= 0.0, v, 0.2 * v)
                            acc = acc + att_v[slw] * m
                    for sh in (8, 4, 2, 1):
                        acc = acc + acc.at[lane ^ sh].get(
                            mode="promise_in_bounds")
                    onehot = jnp.where(lane == t, 1.0, 0.0)
                    lgv = lgv + acc * onehot
                lg[pl.ds(g * L, L)] = lgv
                return 0

            lax.fori_loop(0, C // L, grp, 0)

            def expg(g, _):
                sl = pl.ds(g * L, L)
                lg[sl] = jnp.exp(lg[sl])
                return 0

            lax.fori_loop(0, C // L, expg, 0)
            pltpu.sync_copy(lg, ex_out.at[pl.ds(k * C, C)])

        pl.when(wid < NCHUNK)(lambda: issue(wid, 0))

        def outer(jj, _):
            for b in (0, 1):
                k = (2 * jj + b) * NW + wid
                kn = k + NW
                pl.when(kn < NCHUNK)(lambda kn=kn, b=b: issue(kn, 1 - b))
                pl.when(k < NCHUNK)(lambda k=k, b=b: compute(k, b))
            return 0

        lax.fori_loop(0, NJ2, outer, 0)

    def one_set():
        return [
            pltpu.VMEM((2, C), jnp.int32),
            pltpu.VMEM((C,), jnp.int32),
            pltpu.VMEM((C,), jnp.int32),
            pltpu.VMEM((C,), jnp.float32),
            pltpu.VMEM((C,), jnp.float32),
            pltpu.VMEM((C, 128), jnp.float32),
            pltpu.VMEM((C, 128), jnp.float32),
            pltpu.VMEM((C, 128), jnp.float32),
            pltpu.VMEM((C, 128), jnp.float32),
        ]

    fwidth = npairs * 128
    scratch = one_set() + one_set() + [
        pltpu.VMEM((fwidth,), jnp.float32),
        pltpu.VMEM((fwidth,), jnp.float32),
        pltpu.SemaphoreType.DMA((8,)),
    ]
    return pl.kernel(
        body,
        out_type=jax.ShapeDtypeStruct((E,), jnp.float32),
        mesh=_MESH,
        scratch_types=scratch,
        compiler_params=pltpu.CompilerParams(use_tc_tiling_on_sc=False),
    )


def _make_accum(split_edges, idx_core_off, C):
    NCHUNK = E // C
    RPT = N // NS
    ZR = 25

    def body(tab, edata, ex, tf_out, td_out, *refs):
        nper = 5
        sets = [refs[i * nper:(i + 1) * nper] for i in range(2)]
        zbuf, zbufd, t_spf, t_spd, sems = refs[2 * nper:]
        c = lax.axis_index("c")
        s = lax.axis_index("s")
        owner = s * NC + c if split_edges else s
        nown = NW if split_edges else NS
        nj2 = (NCHUNK // nown + 2) // 2

        def zrow(r, _):
            for kk in range(128 // L):
                zbuf[r, pl.ds(kk * L, L)] = jnp.zeros((L,), jnp.float32)
            zbufd[r, pl.ds(0, L)] = jnp.zeros((L,), jnp.float32)
            return 0

        lax.fori_loop(0, ZR, zrow, 0)
        for z in range(RPT // ZR):
            pltpu.sync_copy(zbuf, t_spf.at[pl.ds(s * RPT + z * ZR, ZR)])
            pltpu.sync_copy(zbufd, t_spd.at[pl.ds(s * RPT + z * ZR, ZR)])
        plsc.subcore_barrier()

        lane = lax.iota(jnp.int32, L)
        onehot0 = jnp.where(lane == 0, 1.0, 0.0)

        def drain(s_):
            eb, siadj, exv, a0, exb = sets[s_]
            pltpu.make_async_copy(a0, t_spf.at[eb.at[1]],
                                  sems.at[2 + 2 * s_]).wait()
            pltpu.make_async_copy(exb, t_spd.at[eb.at[1]],
                                  sems.at[3 + 2 * s_]).wait()

        def issue(k, s_):
            eb, siadj, exv, a0, exb = sets[s_]
            pl.when(k >= 2 * nown)(lambda: drain(s_))
            off = k * C
            pltpu.sync_copy(edata.at[:, pl.ds(off, C)], eb)
            pltpu.sync_copy(ex.at[pl.ds(off, C)], exv)
            for g in range(C // L):
                sl = pl.ds(g * L, L)
                siadj[sl] = eb[0, sl] + c * idx_core_off
            pltpu.async_copy(tab.at[siadj], a0, sems.at[s_])

        def compute(k, s_):
            eb, siadj, exv, a0, exb = sets[s_]
            pltpu.make_async_copy(tab.at[siadj], a0, sems.at[s_]).wait()

            def grp(g, _):
                exvv = exv[pl.ds(g * L, L)]
                for t in range(L):
                    e = g * L + t
                    x = exvv[t]
                    exb[e, pl.ds(0, L)] = x * onehot0
                    for kk in range(128 // L):
                        sl = pl.ds(kk * L, L)
                        a0[e, sl] = x * a0[e, sl]
                return 0

            lax.fori_loop(0, C // L, grp, 0)
            pltpu.async_copy(a0, t_spf.at[eb.at[1]], sems.at[2 + 2 * s_],
                             add=True)
            pltpu.async_copy(exb, t_spd.at[eb.at[1]], sems.at[3 + 2 * s_],
                             add=True)

        pl.when(owner < NCHUNK)(lambda: issue(owner, 0))

        def outer(jj, _):
            for b in (0, 1):
                k = (2 * jj + b) * nown + owner
                kn = k + nown
                pl.when(kn < NCHUNK)(lambda kn=kn, b=b: issue(kn, 1 - b))
                pl.when(k < NCHUNK)(lambda k=k, b=b: compute(k, b))
            return 0

        lax.fori_loop(0, nj2, outer, 0)
        drain(0)
        drain(1)
        plsc.subcore_barrier()
        pltpu.sync_copy(t_spf.at[pl.ds(s * RPT, RPT)],
                        tf_out.at[pl.ds(c * N + s * RPT, RPT)])
        pltpu.sync_copy(t_spd.at[pl.ds(s * RPT, RPT)],
                        td_out.at[pl.ds(c * N + s * RPT, RPT)])

    def one_set():
        return [
            pltpu.VMEM((2, C), jnp.int32),
            pltpu.VMEM((C,), jnp.int32),
            pltpu.VMEM((C,), jnp.float32),
            pltpu.VMEM((C, 128), jnp.float32),
            pltpu.VMEM((C, L), jnp.float32),
        ]

    scratch = one_set() + one_set() + [
        pltpu.VMEM((ZR, 128), jnp.float32),
        pltpu.VMEM((ZR, L), jnp.float32),
        pltpu.VMEM_SHARED((N, 128), jnp.float32),
        pltpu.VMEM_SHARED((N, L), jnp.float32),
        pltpu.SemaphoreType.DMA((6,)),
    ]
    return pl.kernel(
        body,
        out_type=[jax.ShapeDtypeStruct((2 * N, 128), jnp.float32),
                  jax.ShapeDtypeStruct((2 * N, L), jnp.float32)],
        mesh=_MESH,
        scratch_types=scratch,
        compiler_params=pltpu.CompilerParams(use_tc_tiling_on_sc=False),
    )


def _mid_body(tfa_ref, tfb_ref, td_ref, w_ref, b_ref, oxl_ref, oxr_ref):
    den = td_ref[:, 0:1] + 1e-16
    hlo = tfa_ref[...] / den
    hhi = tfb_ref[...] / den
    h = jnp.concatenate([hlo, hhi], axis=1) + b_ref[...]
    h = jnp.where(h > 0.0, h, jnp.exp(jnp.minimum(h, 0.0)) - 1.0)
    res = jnp.dot(h, w_ref[...], preferred_element_type=jnp.float32)
    oxl_ref[...] = res[:, :128]
    oxr_ref[...] = res[:, 128:]


def _mid(tf, td, w2cat, b1):
    return pl.pallas_call(
        _mid_body,
        grid=(N // BM,),
        in_specs=[
            pl.BlockSpec((BM, 128), lambda i: (i, 0)),
            pl.BlockSpec((BM, 128), lambda i: (i + N // BM, 0)),
            pl.BlockSpec((BM, L), lambda i: (i, 0)),
            pl.BlockSpec((256, 256), lambda i: (0, 0)),
            pl.BlockSpec((1, 256), lambda i: (0, 0)),
        ],
        out_specs=[
            pl.BlockSpec((BM, 128), lambda i: (i, 0)),
            pl.BlockSpec((BM, 128), lambda i: (i, 0)),
        ],
        out_shape=[
            jax.ShapeDtypeStruct((N, 128), jnp.float32),
            jax.ShapeDtypeStruct((N, 128), jnp.float32),
        ],
    )(tf, tf, td, w2cat, b1.reshape(1, 256))


_BN_INV = 0.9999950000374997


def _final_body(tfa_ref, tfb_ref, tda_ref, tdb_ref, batch_ref, b2_ref,
                fc1w_ref, fc1b_ref, gamma_ref, beta_ref, fc3w_ref, fc3b_ref,
                out_ref, sums_ref, cnt_ref):
    i = pl.program_id(0)

    @pl.when(i == 0)
    def _():
        sums_ref[...] = jnp.zeros_like(sums_ref)
        cnt_ref[...] = jnp.zeros_like(cnt_ref)

    den = tda_ref[:, 0:1] + tdb_ref[:, 0:1] + 1e-16
    h = (tfa_ref[...] + tfb_ref[...]) / den + b2_ref[...]
    h = jnp.where(h > 0.0, h, jnp.exp(jnp.minimum(h, 0.0)) - 1.0)
    ids = lax.broadcasted_iota(jnp.int32, (G, BM), 0)
    onehot = (ids == batch_ref[0]).astype(jnp.float32)
    sums_ref[...] += jnp.dot(onehot, h, preferred_element_type=jnp.float32)
    cnt_ref[...] += jnp.sum(onehot, axis=1, keepdims=True)

    @pl.when(i == N // BM - 1)
    def _():
        pooled = sums_ref[...] / jnp.maximum(cnt_ref[:, 0:1], 1.0)
        z = jnp.dot(pooled, fc1w_ref[...], preferred_element_type=jnp.float32)
        z = jnp.maximum(z + fc1b_ref[...], 0.0)
        z = gamma_ref[...] * z * _BN_INV + beta_ref[...]
        out_ref[...] = (
            jnp.dot(z, fc3w_ref[...], preferred_element_type=jnp.float32)
            + fc3b_ref[...])


def _final(tf2, td2, batch3d, b2, fc1W, fc1b, gamma, beta, fc3W, fc3b):
    return pl.pallas_call(
        _final_body,
        grid=(N // BM,),
        in_specs=[
            pl.BlockSpec((BM, 128), lambda i: (i, 0)),
            pl.BlockSpec((BM, 128), lambda i: (i + N // BM, 0)),
            pl.BlockSpec((BM, L), lambda i: (i, 0)),
            pl.BlockSpec((BM, L), lambda i: (i + N // BM, 0)),
            pl.BlockSpec((1, 1, BM), lambda i: (i, 0, 0)),
            pl.BlockSpec((1, 128), lambda i: (0, 0)),
            pl.BlockSpec((128, 128), lambda i: (0, 0)),
            pl.BlockSpec((1, 128), lambda i: (0, 0)),
            pl.BlockSpec((1, 128), lambda i: (0, 0)),
            pl.BlockSpec((1, 128), lambda i: (0, 0)),
            pl.BlockSpec((128, 1), lambda i: (0, 0)),
            pl.BlockSpec((1, 1), lambda i: (0, 0)),
        ],
        out_specs=pl.BlockSpec((G, 1), lambda i: (0, 0)),
        out_shape=jax.ShapeDtypeStruct((G, 1), jnp.float32),
        scratch_shapes=[
            pltpu.VMEM((G, 128), jnp.float32),
            pltpu.VMEM((G, 1), jnp.float32),
        ],
    )(tf2, tf2, td2, td2, batch3d, b2.reshape(1, 128), fc1W,
      fc1b.reshape(1, 128), gamma.reshape(1, 128), beta.reshape(1, 128),
      fc3W, fc3b.reshape(1, 1))


_edge_ex2 = _make_edge_ex(2)
_edge_ex1 = _make_edge_ex(1)
_accum_l1 = _make_accum(split_edges=False, idx_core_off=N, C=80)
_accum_l2 = _make_accum(split_edges=True, idx_core_off=0, C=80)


def kernel(x, edge_index, batch, edge_attr, Wl1, Wr1, We1, att1, b1,
           Wl2, Wr2, We2, att2, b2, fc1W, fc1b, gamma, beta, fc3W, fc3b):
    ea = edge_attr[:, 0]

    wcat1 = jnp.concatenate(
        [Wl1[:, :128], Wr1[:, :128], Wl1[:, 128:], Wr1[:, 128:]], axis=1)
    xl1, xr1 = _matmul1(x, wcat1)

    ex1 = _edge_ex2(xl1, xr1, edge_index, ea, att1, We1[0])
    tf1, td1 = _accum_l1(xl1, edge_index, ex1)

    w2cat = jnp.concatenate([Wl2, Wr2], axis=1)
    xl2, xr2 = _mid(tf1, td1, w2cat, b1)

    ex2 = _edge_ex1(xl2, xr2, edge_index, ea, att2, We2[0])
    tf2, td2 = _accum_l2(xl2, edge_index, ex2)

    return _final(tf2, td2, batch.reshape(N // BM, 1, BM), b2, fc1W, fc1b,
                  gamma, beta, fc3W, fc3b)
